# baseline (speedup 1.0000x reference)
import functools

import jax
import jax.numpy as jnp
from jax import lax
from jax.experimental import pallas as pl
from jax.experimental.pallas import tpu as pltpu
from jax.experimental.pallas import tpu_sc as plsc

M = 256
N = 256
DIM = 64
R = M * N
DECAY = 0.999
ALPHA = 0.3
SIGMA = max(M, N) / 2.0

NC = 2
NS = 16
NW = NC * NS
ROWS_W = R // NW
CH = 256
NCHUNK = ROWS_W // CH
GRP = CH // 16

_mesh = plsc.VectorSubcoreMesh(
    core_axis_name="c", subcore_axis_name="s", num_cores=NC, num_subcores=NS
)

_params = pltpu.CompilerParams(use_tc_tiling_on_sc=True)

_DNUMS = lax.GatherDimensionNumbers(
    offset_dims=(), collapsed_slice_dims=(0,), start_index_map=(0,)
)


def _perm(v, idx):
    return lax.gather(
        v, idx[:, None], _DNUMS, (1,), mode=lax.GatherScatterMode.PROMISE_IN_BOUNDS
    )


def _tree(v, op):
    i = lax.iota(jnp.int32, 16)
    for sh in (8, 4, 2, 1):
        v = op(v, _perm(v, i ^ sh))
    return v


def _worker_base():
    wid = lax.axis_index("s") * NC + lax.axis_index("c")
    return wid, wid * ROWS_W


@functools.partial(
    pl.kernel,
    out_type=[
        jax.ShapeDtypeStruct((NW, 16), jnp.float32),
        jax.ShapeDtypeStruct((NW, 16), jnp.int32),
    ],
    mesh=_mesh,
    scratch_types=[
        pltpu.VMEM((DIM,), jnp.float32),
        pltpu.VMEM((CH, DIM), jnp.float32),
        pltpu.VMEM((CH, DIM), jnp.float32),
        pltpu.VMEM((1, 16), jnp.float32),
        pltpu.VMEM((1, 16), jnp.int32),
        pltpu.SemaphoreType.DMA,
        pltpu.SemaphoreType.DMA,
    ],
    compiler_params=_params,
)
def _bmu_kernel(x_hbm, w_hbm, mind_hbm, mini_hbm, xv, b0, b1, outd, outi, s0, s1):
    wid, base = _worker_base()
    pltpu.sync_copy(x_hbm, xv)
    x2 = [2.0 * xv[pl.ds(16 * q, 16)] for q in range(4)]

    bufs = [b0, b1]
    sems = [s0, s1]
    cps = [None, None]
    cps[0] = pltpu.async_copy(w_hbm.at[pl.ds(base, CH)], bufs[0], sems[0])

    md = jnp.full((16,), jnp.inf, jnp.float32)
    mi = jnp.zeros((16,), jnp.int32)
    for c in range(NCHUNK):
        cps[c % 2].wait()
        if c + 1 < NCHUNK:
            cps[(c + 1) % 2] = pltpu.async_copy(
                w_hbm.at[pl.ds(base + (c + 1) * CH, CH)],
                bufs[(c + 1) % 2],
                sems[(c + 1) % 2],
            )
        cur = bufs[c % 2]

        def grp(g, carry, cur=cur, c=c):
            md, mi = carry
            for j in range(16):
                r = g * 16 + j
                p = None
                for q in range(4):
                    wq = cur[r, pl.ds(16 * q, 16)]
                    t = wq * (wq - x2[q])
                    p = t if p is None else p + t
                s = _tree(p, jnp.add)
                bet = s < md
                md = jnp.where(bet, s, md)
                rglob = base + c * CH + r
                mi = jnp.where(bet, jnp.full((16,), rglob, jnp.int32), mi)
            return md, mi

        md, mi = lax.fori_loop(0, GRP, grp, (md, mi))

    outd[0] = md
    outi[0] = mi
    pltpu.sync_copy(outd, mind_hbm.at[pl.ds(wid, 1)])
    pltpu.sync_copy(outi, mini_hbm.at[pl.ds(wid, 1)])


@functools.partial(
    pl.kernel,
    out_type=jax.ShapeDtypeStruct((R, DIM), jnp.float32),
    mesh=_mesh,
    scratch_types=[
        pltpu.VMEM((DIM,), jnp.float32),
        pltpu.VMEM((2, 16), jnp.float32),
        pltpu.VMEM((NW, 16), jnp.float32),
        pltpu.VMEM((NW, 16), jnp.int32),
        pltpu.VMEM((CH, DIM), jnp.float32),
        pltpu.VMEM((CH, DIM), jnp.float32),
        pltpu.VMEM((CH, DIM), jnp.float32),
        pltpu.SemaphoreType.DMA,
        pltpu.SemaphoreType.DMA,
        pltpu.SemaphoreType.DMA,
        pltpu.SemaphoreType.DMA,
        pltpu.SemaphoreType.DMA,
        pltpu.SemaphoreType.DMA,
    ],
    compiler_params=_params,
)
def _update_kernel(
    x_hbm, w_hbm, p_hbm, mind_hbm, mini_hbm, out_hbm,
    xv, pv, mdv, miv, b0, b1, b2, si0, si1, si2, so0, so1, so2,
):
    wid, base = _worker_base()
    pltpu.sync_copy(x_hbm, xv)
    pltpu.sync_copy(p_hbm, pv)
    pltpu.sync_copy(mind_hbm, mdv)
    pltpu.sync_copy(mini_hbm, miv)

    xs = [xv[pl.ds(16 * q, 16)] for q in range(4)]
    av = pv[0]
    cv = pv[1]

    bd = mdv[0]
    bi = miv[0]
    for j in range(1, NW):
        dv = mdv[j]
        iv = miv[j]
        bet = (dv < bd) | ((dv == bd) & (iv < bi))
        bd = jnp.where(bet, dv, bd)
        bi = jnp.where(bet, iv, bi)
    m = _tree(bd, jnp.minimum)
    cand = jnp.where(bd == m, bi, jnp.int32(1 << 30))
    bmu = _tree(cand, jnp.minimum)
    bxv = bmu & (M - 1)
    byv = bmu >> 8

    bufs = [b0, b1, b2]
    isems = [si0, si1, si2]
    osems = [so0, so1, so2]
    icp = [None] * NCHUNK
    ocp = [None] * NCHUNK
    icp[0] = pltpu.async_copy(w_hbm.at[pl.ds(base, CH)], bufs[0], isems[0])
    icp[1] = pltpu.async_copy(w_hbm.at[pl.ds(base + CH, CH)], bufs[1], isems[1])

    for c in range(NCHUNK):
        icp[c].wait()
        buf = bufs[c % 3]

        def grp(g, carry, buf=buf, c=c):
            riota = base + c * CH + g * 16 + lax.iota(jnp.int32, 16)
            dx = (riota & (M - 1)) - bxv
            dy = (riota >> 8) - byv
            d2f = (dx * dx + dy * dy).astype(jnp.float32)
            lrv = av * jnp.exp(d2f * cv)
            for j in range(16):
                r = g * 16 + j
                ls = _perm(lrv, jnp.full((16,), j, jnp.int32))
                for q in range(4):
                    wq = buf[r, pl.ds(16 * q, 16)]
                    buf[r, pl.ds(16 * q, 16)] = wq + ls * (xs[q] - wq)
            return carry

        lax.fori_loop(0, GRP, grp, 0)
        ocp[c] = pltpu.async_copy(
            buf, out_hbm.at[pl.ds(base + c * CH, CH)], osems[c % 3]
        )
        if c + 2 < NCHUNK:
            if c >= 1:
                ocp[c - 1].wait()
            icp[c + 2] = pltpu.async_copy(
                w_hbm.at[pl.ds(base + (c + 2) * CH, CH)],
                bufs[(c + 2) % 3],
                isems[(c + 2) % 3],
            )

    ocp[NCHUNK - 3].wait()
    ocp[NCHUNK - 2].wait()
    ocp[NCHUNK - 1].wait()


def kernel(x, step, weights, loc_x, loc_y):
    decay = DECAY ** step
    alpha_op = ALPHA * decay
    sigma_op = SIGMA * decay
    coef = -1.0 / (2.0 * sigma_op * sigma_op)
    params = jnp.stack(
        [
            jnp.full((16,), alpha_op, jnp.float32),
            jnp.full((16,), coef, jnp.float32),
        ]
    )
    mind, mini = _bmu_kernel(x, weights)
    return _update_kernel(x, weights, params, mind, mini)


# speedup vs baseline: 2.0510x; 2.0510x over previous
import functools

import jax
import jax.numpy as jnp
from jax import lax
from jax.experimental import pallas as pl
from jax.experimental.pallas import tpu as pltpu
from jax.experimental.pallas import tpu_sc as plsc

M = 256
N = 256
DIM = 64
R = M * N
DECAY = 0.999
ALPHA = 0.3
SIGMA = max(M, N) / 2.0

NC = 2
NS = 16
NW = NC * NS
UNITS_W = R // NW
CHU = 512
NCH = UNITS_W // CHU
NSLOT = 4
BLK = NSLOT * 16
NBLK = CHU // BLK
CHU2 = 256
NCH2 = UNITS_W // CHU2
NSLOT2 = 8
BLK2 = NSLOT2 * 16
NBLK2 = CHU2 // BLK2

_mesh = plsc.VectorSubcoreMesh(
    core_axis_name="c", subcore_axis_name="s", num_cores=NC, num_subcores=NS
)

_DNUMS = lax.GatherDimensionNumbers(
    offset_dims=(), collapsed_slice_dims=(0,), start_index_map=(0,)
)


def _perm(v, idx):
    return lax.gather(
        v, idx[:, None], _DNUMS, (1,), mode=lax.GatherScatterMode.PROMISE_IN_BOUNDS
    )


def _splat(v, j):
    return _perm(v, jnp.full((16,), j, jnp.int32))


def _tree(v, op):
    i = lax.iota(jnp.int32, 16)
    for sh in (8, 4, 2, 1):
        v = op(v, _perm(v, i ^ sh))
    return v


def _worker_base():
    wid = lax.axis_index("s") * NC + lax.axis_index("c")
    return wid, wid * UNITS_W


@functools.partial(
    pl.kernel,
    out_type=[
        jax.ShapeDtypeStruct((NW, 16), jnp.float32),
        jax.ShapeDtypeStruct((NW, 16), jnp.int32),
    ],
    mesh=_mesh,
    scratch_types=[
        pltpu.VMEM((DIM,), jnp.float32),
        pltpu.VMEM((DIM, CHU), jnp.float32),
        pltpu.VMEM((DIM, CHU), jnp.float32),
        pltpu.VMEM((DIM, CHU), jnp.float32),
        pltpu.VMEM((1, 16), jnp.float32),
        pltpu.VMEM((1, 16), jnp.int32),
        pltpu.SemaphoreType.DMA,
        pltpu.SemaphoreType.DMA,
        pltpu.SemaphoreType.DMA,
    ],
)
def _bmu_kernel(
    x_hbm, wt_hbm, mind_hbm, mini_hbm, xv, b0, b1, b2, outd, outi, s0, s1, s2
):
    wid, ubase = _worker_base()
    pltpu.sync_copy(x_hbm, xv)
    x2q = [2.0 * xv[pl.ds(16 * q, 16)] for q in range(4)]

    bufs = [b0, b1, b2]
    sems = [s0, s1, s2]
    cps = [None] * NCH
    cps[0] = pltpu.async_copy(wt_hbm.at[:, pl.ds(ubase, CHU)], bufs[0], sems[0])
    cps[1] = pltpu.async_copy(wt_hbm.at[:, pl.ds(ubase + CHU, CHU)], bufs[1], sems[1])

    md = [jnp.full((16,), jnp.inf, jnp.float32) for _ in range(NSLOT)]
    mi = [jnp.zeros((16,), jnp.int32) for _ in range(NSLOT)]
    iota = lax.iota(jnp.int32, 16)

    for c in range(NCH):
        cps[c].wait()
        if c + 2 < NCH:
            cps[c + 2] = pltpu.async_copy(
                wt_hbm.at[:, pl.ds(ubase + (c + 2) * CHU, CHU)],
                bufs[(c + 2) % 3],
                sems[(c + 2) % 3],
            )
        cur = bufs[c % 3]

        def blk(b, carry, cur=cur, c=c):
            md = list(carry[:NSLOT])
            mi = list(carry[NSLOT:])
            acc = [None] * NSLOT
            for d in range(DIM):
                xsp = _splat(x2q[d // 16], d % 16)
                for s in range(NSLOT):
                    wv = cur[d, pl.ds(b * BLK + s * 16, 16)]
                    t = wv * (wv - xsp)
                    acc[s] = t if acc[s] is None else acc[s] + t
            for s in range(NSLOT):
                uvec = ubase + c * CHU + b * BLK + s * 16 + iota
                bet = acc[s] < md[s]
                md[s] = jnp.where(bet, acc[s], md[s])
                mi[s] = jnp.where(bet, uvec, mi[s])
            return tuple(md) + tuple(mi)

        out = lax.fori_loop(0, NBLK, blk, tuple(md) + tuple(mi))
        md = list(out[:NSLOT])
        mi = list(out[NSLOT:])

    bd, bi = md[0], mi[0]
    for s in range(1, NSLOT):
        bet = (md[s] < bd) | ((md[s] == bd) & (mi[s] < bi))
        bd = jnp.where(bet, md[s], bd)
        bi = jnp.where(bet, mi[s], bi)
    outd[0] = bd
    outi[0] = bi
    pltpu.sync_copy(outd, mind_hbm.at[pl.ds(wid, 1)])
    pltpu.sync_copy(outi, mini_hbm.at[pl.ds(wid, 1)])


@functools.partial(
    pl.kernel,
    out_type=jax.ShapeDtypeStruct((DIM, R), jnp.float32),
    mesh=_mesh,
    scratch_types=[
        pltpu.VMEM((DIM,), jnp.float32),
        pltpu.VMEM((2, 16), jnp.float32),
        pltpu.VMEM((NW, 16), jnp.float32),
        pltpu.VMEM((NW, 16), jnp.int32),
        pltpu.VMEM((DIM, CHU2), jnp.float32),
        pltpu.VMEM((DIM, CHU2), jnp.float32),
        pltpu.VMEM((DIM, CHU2), jnp.float32),
        pltpu.VMEM((DIM, CHU2), jnp.float32),
        pltpu.VMEM((DIM, CHU2), jnp.float32),
        pltpu.VMEM((DIM * 16,), jnp.float32),
        pltpu.SemaphoreType.DMA,
        pltpu.SemaphoreType.DMA,
        pltpu.SemaphoreType.DMA,
        pltpu.SemaphoreType.DMA,
        pltpu.SemaphoreType.DMA,
    ],
)
def _update_kernel(
    x_hbm, wt_hbm, p_hbm, mind_hbm, mini_hbm, out_hbm,
    xv, pv, mdv, miv, i0, i1, i2, o0, o1, xt, si0, si1, si2, so0, so1,
):
    wid, ubase = _worker_base()
    pltpu.sync_copy(x_hbm, xv)
    pltpu.sync_copy(p_hbm, pv)
    pltpu.sync_copy(mind_hbm, mdv)
    pltpu.sync_copy(mini_hbm, miv)

    x1q = [xv[pl.ds(16 * q, 16)] for q in range(4)]
    for d in range(DIM):
        xt[pl.ds(d * 16, 16)] = _splat(x1q[d // 16], d % 16)
    av = pv[0]
    cv = pv[1]

    bd = mdv[0]
    bi = miv[0]
    for j in range(1, NW):
        dv = mdv[j]
        iv = miv[j]
        bet = (dv < bd) | ((dv == bd) & (iv < bi))
        bd = jnp.where(bet, dv, bd)
        bi = jnp.where(bet, iv, bi)
    m = _tree(bd, jnp.minimum)
    cand = jnp.where(bd == m, bi, jnp.int32(1 << 30))
    bmu = _tree(cand, jnp.minimum)
    bxv = bmu & (M - 1)
    byv = bmu >> 8
    iota = lax.iota(jnp.int32, 16)

    ibufs = [i0, i1, i2]
    obufs = [o0, o1]
    isems = [si0, si1, si2]
    osems = [so0, so1]
    icp = [None] * NCH2
    ocp = [None] * NCH2
    icp[0] = pltpu.async_copy(wt_hbm.at[:, pl.ds(ubase, CHU2)], ibufs[0], isems[0])
    icp[1] = pltpu.async_copy(
        wt_hbm.at[:, pl.ds(ubase + CHU2, CHU2)], ibufs[1], isems[1]
    )

    for c in range(NCH2):
        icp[c].wait()
        if c + 2 < NCH2:
            icp[c + 2] = pltpu.async_copy(
                wt_hbm.at[:, pl.ds(ubase + (c + 2) * CHU2, CHU2)],
                ibufs[(c + 2) % 3],
                isems[(c + 2) % 3],
            )
        if c >= 2:
            ocp[c - 2].wait()
        cur = ibufs[c % 3]
        ob = obufs[c % 2]

        def blk(b, carry, cur=cur, ob=ob, c=c):
            lr = []
            for s in range(NSLOT2):
                uvec = ubase + c * CHU2 + b * BLK2 + s * 16 + iota
                dx = (uvec & (M - 1)) - bxv
                dy = (uvec >> 8) - byv
                d2f = (dx * dx + dy * dy).astype(jnp.float32)
                lr.append(av * jnp.exp(d2f * cv))

            def dbody(d, carry2, b=b, cur=cur, ob=ob, lr=lr):
                xsp = xt[pl.ds(d * 16, 16)]
                for s in range(NSLOT2):
                    wv = cur[d, pl.ds(b * BLK2 + s * 16, 16)]
                    ob[d, pl.ds(b * BLK2 + s * 16, 16)] = wv + lr[s] * (xsp - wv)
                return carry2

            lax.fori_loop(0, DIM, dbody, 0)
            return carry

        lax.fori_loop(0, NBLK2, blk, 0)
        ocp[c] = pltpu.async_copy(
            ob, out_hbm.at[:, pl.ds(ubase + c * CHU2, CHU2)], osems[c % 2]
        )

    ocp[NCH2 - 2].wait()
    ocp[NCH2 - 1].wait()


def kernel(x, step, weights, loc_x, loc_y):
    decay = DECAY ** step
    alpha_op = ALPHA * decay
    sigma_op = SIGMA * decay
    coef = -1.0 / (2.0 * sigma_op * sigma_op)
    params = jnp.stack(
        [
            jnp.full((16,), alpha_op, jnp.float32),
            jnp.full((16,), coef, jnp.float32),
        ]
    )
    wt = weights.T
    mind, mini = _bmu_kernel(x, wt)
    out_t = _update_kernel(x, wt, params, mind, mini)
    return out_t.T
